# Initial kernel scaffold; baseline (speedup 1.0000x reference)
import jax
import jax.numpy as jnp
from jax.experimental import pallas as pl


def kernel(x, Wg, bg, W1, b1, W2, b2):
    raise NotImplementedError("write your pallas kernel here")


# speedup vs baseline: 135.7040x; 135.7040x over previous
import functools

import jax
import jax.numpy as jnp
from jax import lax
from jax.experimental import pallas as pl
from jax.experimental.pallas import tpu as pltpu
from jax.experimental.pallas import tpu_sc as plsc

N, D, H, O, E, K = 16384, 1024, 1024, 1024, 64, 2
M = N * K
BM = 256
MPAD = M + E * BM
T = MPAD // BM
BN_R = 2048
BN_C = 1024

SC_NC, SC_NS = 2, 16
SC_NW = SC_NC * SC_NS
SC_CHUNK = 64


def _router_body(x_ref, wg_ref, bg_ref, w_ref, i_ref):
    logits = jnp.dot(x_ref[...], wg_ref[...]) + bg_ref[...]
    m = jnp.max(logits, axis=1, keepdims=True)
    p = jnp.exp(logits - m)
    probs = p / jnp.sum(p, axis=1, keepdims=True)
    ids = lax.broadcasted_iota(jnp.int32, (BN_R, E), 1)
    w1 = jnp.max(probs, axis=1, keepdims=True)
    i1 = jnp.min(jnp.where(probs == w1, ids, E), axis=1, keepdims=True)
    masked = jnp.where(ids == i1, -jnp.inf, probs)
    w2 = jnp.max(masked, axis=1, keepdims=True)
    i2 = jnp.min(jnp.where(masked == w2, ids, E), axis=1, keepdims=True)
    w_ref[...] = jnp.concatenate([w1, w2], axis=1)
    i_ref[...] = jnp.concatenate([i1, i2], axis=1)


def _router(x, Wg, bg):
    return pl.pallas_call(
        _router_body,
        grid=(N // BN_R,),
        in_specs=[
            pl.BlockSpec((BN_R, D), lambda i: (i, 0)),
            pl.BlockSpec((D, E), lambda i: (0, 0)),
            pl.BlockSpec((1, E), lambda i: (0, 0)),
        ],
        out_specs=[
            pl.BlockSpec((BN_R, K), lambda i: (i, 0)),
            pl.BlockSpec((BN_R, K), lambda i: (i, 0)),
        ],
        out_shape=[
            jax.ShapeDtypeStruct((N, K), jnp.float32),
            jax.ShapeDtypeStruct((N, K), jnp.int32),
        ],
    )(x, Wg, bg.reshape(1, E))


def _sc_mesh():
    return plsc.VectorSubcoreMesh(core_axis_name="c", subcore_axis_name="s")


def _dispatch_gather(tok, dest, x):
    rows_per_w = M // SC_NW
    n_chunks = rows_per_w // SC_CHUNK

    @functools.partial(
        pl.kernel,
        mesh=_sc_mesh(),
        out_type=jax.ShapeDtypeStruct((MPAD, D), jnp.float32),
        scratch_types=[
            pltpu.VMEM((SC_CHUNK,), jnp.int32),
            pltpu.VMEM((SC_CHUNK,), jnp.int32),
            pltpu.VMEM((SC_CHUNK, D), jnp.float32),
            pltpu.SemaphoreType.DMA,
        ],
    )
    def k(tok_hbm, dest_hbm, x_hbm, xs_hbm, tok_v, dest_v, buf, sem):
        wid = lax.axis_index("s") * SC_NC + lax.axis_index("c")
        base = wid * rows_per_w

        def body(i, carry):
            b = base + i * SC_CHUNK
            pltpu.sync_copy(tok_hbm.at[pl.ds(b, SC_CHUNK)], tok_v)
            pltpu.sync_copy(dest_hbm.at[pl.ds(b, SC_CHUNK)], dest_v)
            pltpu.async_copy(x_hbm.at[tok_v], buf, sem).wait()
            pltpu.async_copy(buf, xs_hbm.at[dest_v], sem).wait()
            return carry

        lax.fori_loop(0, n_chunks, body, 0)

    return k(tok, dest, x)


def _unsort_gather(posmap, ys):
    rows_per_w = M // SC_NW
    n_chunks = rows_per_w // SC_CHUNK

    @functools.partial(
        pl.kernel,
        mesh=_sc_mesh(),
        out_type=jax.ShapeDtypeStruct((M, O), jnp.float32),
        scratch_types=[
            pltpu.VMEM((SC_CHUNK,), jnp.int32),
            pltpu.VMEM((SC_CHUNK, O), jnp.float32),
            pltpu.SemaphoreType.DMA,
        ],
    )
    def k(pos_hbm, ys_hbm, yu_hbm, pos_v, buf, sem):
        wid = lax.axis_index("s") * SC_NC + lax.axis_index("c")
        base = wid * rows_per_w

        def body(i, carry):
            b = base + i * SC_CHUNK
            pltpu.sync_copy(pos_hbm.at[pl.ds(b, SC_CHUNK)], pos_v)
            pltpu.async_copy(ys_hbm.at[pos_v], buf, sem).wait()
            pltpu.sync_copy(buf, yu_hbm.at[pl.ds(b, SC_CHUNK)])
            return carry

        lax.fori_loop(0, n_chunks, body, 0)

    return k(posmap, ys)


def _ffn_body(te_ref, xs_ref, w1_ref, b1_ref, w2_ref, b2_ref, out_ref):
    x = xs_ref[...]
    h = jnp.maximum(
        jnp.dot(x, w1_ref[0], preferred_element_type=jnp.float32) + b1_ref[0],
        0.0)
    out_ref[...] = (
        jnp.dot(h, w2_ref[0], preferred_element_type=jnp.float32) + b2_ref[0])


def _ffn(tile_expert, xs, W1, b1, W2, b2):
    grid_spec = pltpu.PrefetchScalarGridSpec(
        num_scalar_prefetch=1,
        grid=(T,),
        in_specs=[
            pl.BlockSpec((BM, D), lambda t, te: (t, 0)),
            pl.BlockSpec((1, D, H), lambda t, te: (te[t], 0, 0)),
            pl.BlockSpec((1, 1, H), lambda t, te: (te[t], 0, 0)),
            pl.BlockSpec((1, H, O), lambda t, te: (te[t], 0, 0)),
            pl.BlockSpec((1, 1, O), lambda t, te: (te[t], 0, 0)),
        ],
        out_specs=pl.BlockSpec((BM, O), lambda t, te: (t, 0)),
    )
    return pl.pallas_call(
        _ffn_body,
        grid_spec=grid_spec,
        out_shape=jax.ShapeDtypeStruct((MPAD, O), jnp.float32),
    )(tile_expert, xs, W1, b1.reshape(E, 1, H), W2, b2.reshape(E, 1, O))


def _combine_body(yu_ref, w_ref, out_ref):
    y0 = yu_ref[:, 0, :]
    y1 = yu_ref[:, 1, :]
    out_ref[...] = w_ref[:, 0:1] * y0 + w_ref[:, 1:2] * y1


def _combine(yu, wtop):
    return pl.pallas_call(
        _combine_body,
        grid=(N // BN_C,),
        in_specs=[
            pl.BlockSpec((BN_C, K, O), lambda i: (i, 0, 0)),
            pl.BlockSpec((BN_C, K), lambda i: (i, 0)),
        ],
        out_specs=pl.BlockSpec((BN_C, O), lambda i: (i, 0)),
        out_shape=jax.ShapeDtypeStruct((N, O), jnp.float32),
    )(yu, wtop)


def kernel(x, Wg, bg, W1, b1, W2, b2):
    wtop, itop = _router(x, Wg, bg)

    e_flat = itop.reshape(-1)
    perm = jnp.argsort(e_flat)
    e_sorted = e_flat[perm]
    tok_sorted = (perm // K).astype(jnp.int32)

    counts = jnp.zeros((E,), jnp.int32).at[e_flat].add(1)
    group_start = jnp.concatenate(
        [jnp.zeros((1,), jnp.int32), jnp.cumsum(counts)[:-1]])
    padded_counts = ((counts + BM - 1) // BM) * BM
    padded_end = jnp.cumsum(padded_counts)
    padded_start = padded_end - padded_counts

    rank = jnp.arange(M, dtype=jnp.int32) - group_start[e_sorted]
    dest = (padded_start[e_sorted] + rank).astype(jnp.int32)

    tile_expert = jnp.clip(
        jnp.searchsorted(padded_end, jnp.arange(T, dtype=jnp.int32) * BM,
                         side="right"),
        0, E - 1).astype(jnp.int32)

    posmap = jnp.zeros((M,), jnp.int32).at[perm].set(dest)

    xs = _dispatch_gather(tok_sorted, dest, x)
    ys = _ffn(tile_expert, xs, W1, b1, W2, b2)
    yu = _unsort_gather(posmap, ys)
    return _combine(yu.reshape(N, K, O), wtop)
